# Initial kernel scaffold; baseline (speedup 1.0000x reference)
import functools

import jax
import jax.numpy as jnp
from jax import lax
from jax.experimental import pallas as pl
from jax.experimental.pallas import tpu as pltpu
from jax.experimental.pallas import tpu_sc as plsc

NUM_NODES = 100000
EMB = 32
N_EDGES = 1600000
NC, NS, L = 2, 16, 16
HALF = NUM_NODES // NC
ACC_ROWS = 51200
TRASH = HALF
CHUNK = 1024
STREAM = 128
NCHUNK = 100
EPW = NCHUNK * CHUNK
E_PAD = NS * EPW
TBL_ROWS = NUM_NODES + 8
ZROWS = 800
OUT_PW = HALF // NS
OUT_CH = 625


def _propagate(x_pad, rowp, colp, out_rows):
    mesh = plsc.VectorSubcoreMesh(core_axis_name="c", subcore_axis_name="s")

    @functools.partial(
        pl.kernel,
        out_type=jax.ShapeDtypeStruct((out_rows, EMB), jnp.float32),
        mesh=mesh,
        scratch_types=[
            pltpu.VMEM((CHUNK // STREAM, STREAM), jnp.int32),
            pltpu.VMEM((CHUNK // STREAM, STREAM), jnp.int32),
            pltpu.VMEM((CHUNK // STREAM, STREAM), jnp.int32),
            pltpu.VMEM((CHUNK, EMB), jnp.float32),
            pltpu.VMEM((ZROWS, EMB), jnp.float32),
            pltpu.VMEM_SHARED((ACC_ROWS, EMB), jnp.float32),
        ],
    )
    def k(x_hbm, row_hbm, col_hbm, out_hbm, dst_v, src_v, lidx_v, rows_v,
          zero_v, acc):
        c = lax.axis_index("c")
        s = lax.axis_index("s")
        base = c * HALF

        @pl.loop(0, ZROWS)
        def _(r):
            @pl.loop(0, EMB, step=L)
            def _(h):
                zero_v[r, pl.ds(h, L)] = jnp.zeros((L,), jnp.float32)

        span = ACC_ROWS // NS

        @pl.loop(0, span // ZROWS)
        def _(t):
            pltpu.sync_copy(zero_v, acc.at[pl.ds(s * span + t * ZROWS, ZROWS)])

        plsc.subcore_barrier()

        for dst_hbm, src_hbm in ((row_hbm, col_hbm), (col_hbm, row_hbm)):
            @pl.loop(0, NCHUNK)
            def _(ch):
                r0 = s * (EPW // STREAM) + ch * (CHUNK // STREAM)
                pltpu.sync_copy(dst_hbm.at[pl.ds(r0, CHUNK // STREAM)], dst_v)
                pltpu.sync_copy(src_hbm.at[pl.ds(r0, CHUNK // STREAM)], src_v)

                @pl.loop(0, CHUNK // STREAM)
                def _(i):
                    @pl.loop(0, STREAM, step=L)
                    def _(j):
                        d = dst_v[i, pl.ds(j, L)]
                        lu = (d - base).astype(jnp.uint32)
                        lidx_v[i, pl.ds(j, L)] = jnp.minimum(
                            lu, jnp.uint32(TRASH)).astype(jnp.int32)

                @pl.loop(0, CHUNK // STREAM)
                def _(i):
                    rows = rows_v.at[pl.ds(i * STREAM, STREAM)]
                    pltpu.sync_copy(x_hbm.at[src_v.at[i]], rows)
                    pltpu.sync_copy(rows, acc.at[lidx_v.at[i]], add=True)

        plsc.subcore_barrier()

        @pl.loop(0, OUT_PW // OUT_CH)
        def _(t):
            lo = s * OUT_PW + t * OUT_CH
            pltpu.sync_copy(acc.at[pl.ds(lo, OUT_CH)],
                            rows_v.at[pl.ds(0, OUT_CH)])

            @pl.loop(0, OUT_CH)
            def _(r):
                @pl.loop(0, EMB, step=L)
                def _(h):
                    rows_v[r, pl.ds(h, L)] = rows_v[r, pl.ds(h, L)] * 0.5

            pltpu.sync_copy(rows_v.at[pl.ds(0, OUT_CH)],
                            out_hbm.at[pl.ds(base + lo, OUT_CH)])

        if out_rows > NUM_NODES:
            @pl.when(jnp.logical_and(c == NC - 1, s == NS - 1))
            def _():
                pltpu.sync_copy(
                    zero_v.at[pl.ds(0, out_rows - NUM_NODES)],
                    out_hbm.at[pl.ds(NUM_NODES, out_rows - NUM_NODES)])

    return k(x_pad, rowp, colp)


def kernel(edge_index, user_embedding, item_embedding):
    row = edge_index[0].astype(jnp.int32)
    col = edge_index[1].astype(jnp.int32)
    pad = jnp.full((E_PAD - N_EDGES,), NUM_NODES, dtype=jnp.int32)
    rowp = jnp.concatenate([row, pad]).reshape(E_PAD // STREAM, STREAM)
    colp = jnp.concatenate([col, pad]).reshape(E_PAD // STREAM, STREAM)
    x0 = jnp.concatenate([
        user_embedding, item_embedding,
        jnp.zeros((TBL_ROWS - NUM_NODES, EMB), jnp.float32)])
    y1 = _propagate(x0, rowp, colp, TBL_ROWS)
    return _propagate(y1, rowp, colp, NUM_NODES)


# speedup vs baseline: 3.6506x; 3.6506x over previous
import functools

import jax
import jax.numpy as jnp
from jax import lax
from jax.experimental import pallas as pl
from jax.experimental.pallas import tpu as pltpu
from jax.experimental.pallas import tpu_sc as plsc

NUM_NODES = 100000
EMB = 32
N_EDGES = 1600000
NC, NS, L = 2, 16, 16
HALF = NUM_NODES // NC
ACC_ROWS = 50176
TRASH = HALF
CHUNK = 512
STREAM = 128
NCHUNK = 200
EPW = NCHUNK * CHUNK
E_PAD = NS * EPW
TBL_ROWS = NUM_NODES + 8
ZROWS = 392
OUT_CH = 400
OUT_NCH = HALF // OUT_CH


def _propagate(x_pad, rowp, colp, out_rows):
    mesh = plsc.VectorSubcoreMesh(core_axis_name="c", subcore_axis_name="s")

    @functools.partial(
        pl.kernel,
        out_type=jax.ShapeDtypeStruct((out_rows, EMB), jnp.float32),
        mesh=mesh,
        scratch_types=[
            pltpu.VMEM((CHUNK // STREAM, STREAM), jnp.int32),
            pltpu.VMEM((CHUNK // STREAM, STREAM), jnp.int32),
            pltpu.VMEM((CHUNK // STREAM, STREAM), jnp.int32),
            pltpu.VMEM((CHUNK, EMB), jnp.float32),
            pltpu.VMEM_SHARED((ACC_ROWS, EMB), jnp.float32),
        ],
        compiler_params=pltpu.CompilerParams(use_tc_tiling_on_sc=False),
    )
    def k(x_hbm, row_hbm, col_hbm, out_hbm, dst_v, src_v, lidx_v, rows_v,
          acc):
        c = lax.axis_index("c")
        s = lax.axis_index("s")
        base = c * HALF

        @pl.loop(0, CHUNK)
        def _(r):
            @pl.loop(0, EMB, step=L)
            def _(h):
                rows_v[r, pl.ds(h, L)] = jnp.zeros((L,), jnp.float32)

        span = ACC_ROWS // NS

        @pl.loop(0, span // ZROWS)
        def _(t):
            pltpu.sync_copy(rows_v.at[pl.ds(0, ZROWS)],
                            acc.at[pl.ds(s * span + t * ZROWS, ZROWS)])

        plsc.subcore_barrier()

        for dst_hbm, src_hbm in ((row_hbm, col_hbm), (col_hbm, row_hbm)):
            @pl.loop(0, NCHUNK)
            def _(ch):
                r0 = s * (EPW // STREAM) + ch * (CHUNK // STREAM)
                pltpu.sync_copy(dst_hbm.at[pl.ds(r0, CHUNK // STREAM)], dst_v)
                pltpu.sync_copy(src_hbm.at[pl.ds(r0, CHUNK // STREAM)], src_v)

                @pl.loop(0, CHUNK // STREAM)
                def _(i):
                    @pl.loop(0, STREAM, step=L)
                    def _(j):
                        d = dst_v[i, pl.ds(j, L)]
                        lu = (d - base).astype(jnp.uint32)
                        lidx_v[i, pl.ds(j, L)] = jnp.minimum(
                            lu, jnp.uint32(TRASH)).astype(jnp.int32)

                @pl.loop(0, CHUNK // STREAM)
                def _(i):
                    rows = rows_v.at[pl.ds(i * STREAM, STREAM)]
                    pltpu.sync_copy(x_hbm.at[src_v.at[i]], rows)
                    pltpu.sync_copy(rows, acc.at[lidx_v.at[i]], add=True)

        plsc.subcore_barrier()

        @pl.loop(0, (OUT_NCH + NS - 1) // NS)
        def _(t):
            u = s + t * NS

            @pl.when(u < OUT_NCH)
            def _():
                lo = u * OUT_CH
                pltpu.sync_copy(acc.at[pl.ds(lo, OUT_CH)],
                                rows_v.at[pl.ds(0, OUT_CH)])

                @pl.loop(0, OUT_CH)
                def _(r):
                    @pl.loop(0, EMB, step=L)
                    def _(h):
                        rows_v[r, pl.ds(h, L)] = rows_v[r, pl.ds(h, L)] * 0.5

                pltpu.sync_copy(rows_v.at[pl.ds(0, OUT_CH)],
                                out_hbm.at[pl.ds(base + lo, OUT_CH)])

        if out_rows > NUM_NODES:
            @pl.when(jnp.logical_and(c == NC - 1, s == NS - 1))
            def _():
                npad = out_rows - NUM_NODES

                @pl.loop(0, npad)
                def _(r):
                    @pl.loop(0, EMB, step=L)
                    def _(h):
                        rows_v[r, pl.ds(h, L)] = jnp.zeros((L,), jnp.float32)

                pltpu.sync_copy(rows_v.at[pl.ds(0, npad)],
                                out_hbm.at[pl.ds(NUM_NODES, npad)])

    return k(x_pad, rowp, colp)


def kernel(edge_index, user_embedding, item_embedding):
    row = edge_index[0].astype(jnp.int32)
    col = edge_index[1].astype(jnp.int32)
    pad = jnp.full((E_PAD - N_EDGES,), NUM_NODES, dtype=jnp.int32)
    rowp = jnp.concatenate([row, pad]).reshape(E_PAD // STREAM, STREAM)
    colp = jnp.concatenate([col, pad]).reshape(E_PAD // STREAM, STREAM)
    x0 = jnp.concatenate([
        user_embedding, item_embedding,
        jnp.zeros((TBL_ROWS - NUM_NODES, EMB), jnp.float32)])
    y1 = _propagate(x0, rowp, colp, TBL_ROWS)
    return _propagate(y1, rowp, colp, NUM_NODES)
